# baseline (speedup 1.0000x reference)
import functools

import jax
import jax.numpy as jnp
from jax import lax
from jax.experimental import pallas as pl
from jax.experimental.pallas import tpu as pltpu
from jax.experimental.pallas import tpu_sc as plsc

_NC = 2
_NS = 16
_L = 16
_NW = _NC * _NS
_B = 16384
_F = 64
_BPW = _B // _NW
_CH = 128
_NCH = _BPW // _CH
_G = _BPW // _L


def _body(p_ref, poll_ref, pe_hbm, pb_hbm, qe_hbm, qb_hbm, out_hbm,
          idx_p, idx_q, pe_v, qe_v, pb_v, qb_v, out_v, sem):
    c = lax.axis_index("c")
    s = lax.axis_index("s")
    wid = s * _NC + c
    row0 = wid * _NCH
    base = wid * _BPW

    pltpu.sync_copy(p_ref.at[pl.ds(row0, _NCH)], idx_p)
    pltpu.sync_copy(poll_ref.at[pl.ds(row0, _NCH)], idx_q)

    copies = []
    for j in range(_NCH):
        dst = pl.ds(j * _CH, _CH)
        copies.append(pltpu.async_copy(pe_hbm.at[idx_p.at[j]], pe_v.at[dst], sem))
        copies.append(pltpu.async_copy(qe_hbm.at[idx_q.at[j]], qe_v.at[dst], sem))
        copies.append(pltpu.async_copy(pb_hbm.at[idx_p.at[j]], pb_v.at[dst], sem))
        copies.append(pltpu.async_copy(qb_hbm.at[idx_q.at[j]], qb_v.at[dst], sem))
    for cp in copies:
        cp.wait()

    iota = lax.iota(jnp.int32, _L)

    def group(g, carry):
        dots = jnp.zeros((_L,), jnp.float32)
        for r in range(_L):
            row = g * _L + r
            s = None
            for k in range(_F // _L):
                a = pe_v[row, pl.ds(k * _L, _L)]
                b = qe_v[row, pl.ds(k * _L, _L)]
                s = a * b if s is None else s + a * b
            dot = jnp.sum(s)
            dots = jnp.where(iota == r, dot, dots)
        off = pl.ds(g * _L, _L)
        x = dots + pb_v[off] + qb_v[off]
        out_v[off] = 1.0 / (1.0 + jnp.exp(-x))
        return carry

    lax.fori_loop(0, _G, group, 0)

    pltpu.sync_copy(out_v, out_hbm.at[pl.ds(base, _BPW)])


@jax.jit
def kernel(p, poll, p_embed, p_bias, poll_embed, poll_bias):
    p2 = p.astype(jnp.int32).reshape(_NW * _NCH, _CH)
    q2 = poll.astype(jnp.int32).reshape(_NW * _NCH, _CH)
    pb = p_bias.reshape(-1)
    qb = poll_bias.reshape(-1)
    mesh = plsc.VectorSubcoreMesh(core_axis_name="c", subcore_axis_name="s")
    run = pl.kernel(
        _body,
        mesh=mesh,
        compiler_params=pltpu.CompilerParams(
            needs_layout_passes=False, use_tc_tiling_on_sc=False),
        out_type=jax.ShapeDtypeStruct((_B,), jnp.float32),
        scratch_types=[
            pltpu.VMEM((_NCH, _CH), jnp.int32),
            pltpu.VMEM((_NCH, _CH), jnp.int32),
            pltpu.VMEM((_BPW, _F), jnp.float32),
            pltpu.VMEM((_BPW, _F), jnp.float32),
            pltpu.VMEM((_BPW,), jnp.float32),
            pltpu.VMEM((_BPW,), jnp.float32),
            pltpu.VMEM((_BPW,), jnp.float32),
            pltpu.SemaphoreType.DMA,
        ],
    )
    return run(p2, q2, p_embed, pb, poll_embed, qb)
